# Initial kernel scaffold; baseline (speedup 1.0000x reference)
import jax
import jax.numpy as jnp
from jax.experimental import pallas as pl


def kernel(loc_preds, score_preds, gt_data, priors):
    raise NotImplementedError("write your pallas kernel here")


# speedup vs baseline: 5.9024x; 5.9024x over previous
import functools

import jax
import jax.numpy as jnp
from jax.experimental import pallas as pl

_B, _P, _C, _O = 32, 8732, 81, 12
_NEG_POS = 3
_OVTH = 0.5
_CE_BLK = 4096


def _match_body(gt_ref, pri_ref, locp_ref, conf_ref, lp_ref):
    px = pri_ref[0:1, :]
    py = pri_ref[1:2, :]
    pw = pri_ref[2:3, :]
    ph = pri_ref[3:4, :]
    x1 = px - pw * 0.5
    y1 = py - ph * 0.5
    x2 = px + pw * 0.5
    y2 = py + ph * 0.5
    area_p = (x2 - x1) * (y2 - y1)
    iota = jax.lax.broadcasted_iota(jnp.int32, (1, _P), 1)

    best_ov = jnp.full((1, _P), -1.0, jnp.float32)
    best_idx = jnp.zeros((1, _P), jnp.int32)
    bpis = []
    for t in range(_O):
        tx1 = gt_ref[0, t, 0]
        ty1 = gt_ref[0, t, 1]
        tx2 = gt_ref[0, t, 2]
        ty2 = gt_ref[0, t, 3]
        ix = jnp.maximum(jnp.minimum(tx2, x2) - jnp.maximum(tx1, x1), 0.0)
        iy = jnp.maximum(jnp.minimum(ty2, y2) - jnp.maximum(ty1, y1), 0.0)
        inter = ix * iy
        area_t = (tx2 - tx1) * (ty2 - ty1)
        ov = inter / (area_t + area_p - inter)
        m = jnp.max(ov)
        bpis.append(jnp.min(jnp.where(ov == m, iota, _P)))
        upd = ov > best_ov
        best_ov = jnp.where(upd, ov, best_ov)
        best_idx = jnp.where(upd, t, best_idx)
    for t in range(_O):
        hit = iota == bpis[t]
        best_ov = jnp.where(hit, 2.0, best_ov)
        best_idx = jnp.where(hit, t, best_idx)

    conf = jnp.zeros((1, _P), jnp.float32)
    mx1 = jnp.zeros((1, _P), jnp.float32)
    my1 = jnp.zeros((1, _P), jnp.float32)
    mx2 = jnp.zeros((1, _P), jnp.float32)
    my2 = jnp.zeros((1, _P), jnp.float32)
    for t in range(_O):
        sel = (best_idx == t).astype(jnp.float32)
        conf = conf + sel * gt_ref[0, t, 4]
        mx1 = mx1 + sel * gt_ref[0, t, 0]
        my1 = my1 + sel * gt_ref[0, t, 1]
        mx2 = mx2 + sel * gt_ref[0, t, 2]
        my2 = my2 + sel * gt_ref[0, t, 3]
    confi = jnp.where(best_ov < _OVTH, 0, conf.astype(jnp.int32))
    conf_ref[...] = confi.reshape(1, 1, _P)
    posf = (confi > 0).astype(jnp.float32)

    g_cx = ((mx1 + mx2) * 0.5 - px) / (0.1 * pw)
    g_cy = ((my1 + my2) * 0.5 - py) / (0.1 * ph)
    g_w = jnp.log((mx2 - mx1) / pw) / 0.2
    g_h = jnp.log((my2 - my1) / ph) / 0.2
    s = jnp.float32(0.0)
    for d in range(4):
        df = locp_ref[0, d, :].reshape(1, _P) - (g_cx, g_cy, g_w, g_h)[d]
        ad = jnp.abs(df)
        s = s + jnp.sum(jnp.where(ad < 1.0, 0.5 * df * df, ad - 0.5) * posf)
    lp_ref[...] = s.reshape(1, 1, 1)


def _ce_body(x_ref, tgt_ref, ce_ref):
    x = x_ref[...]
    tgt = tgt_ref[...]
    m = jnp.max(x, axis=1, keepdims=True)
    s = jnp.sum(jnp.exp(x - m), axis=1, keepdims=True)
    cls = jax.lax.broadcasted_iota(jnp.int32, x.shape, 1)
    gathered = jnp.sum(jnp.where(cls == tgt, x, 0.0), axis=1, keepdims=True)
    ce_ref[...] = jnp.log(s) + m - gathered


def _sel_body(ce_ref, conf_ref, lp_ref, loc_out, conf_out):
    ce = ce_ref[...]
    pos = conf_ref[...] > 0
    posf = pos.astype(jnp.float32)
    nposi = jnp.sum(pos.astype(jnp.int32), axis=1, keepdims=True)
    total = jnp.sum(nposi)
    k = jnp.minimum(_NEG_POS * nposi, _P - total - 1)
    cl = jnp.where(pos, 0.0, ce)
    clb = jax.lax.bitcast_convert_type(cl, jnp.int32)
    lo = jnp.zeros((_B, 1), jnp.int32)
    for bit in range(30, -1, -1):
        cand = lo + (1 << bit)
        cnt = jnp.sum((clb >= cand).astype(jnp.int32), axis=1, keepdims=True)
        lo = jnp.where(cnt >= k, cand, lo)
    gt = clb > lo
    eq = clb == lo
    cnt_gt = jnp.sum(gt.astype(jnp.float32), axis=1, keepdims=True)
    sum_gt = jnp.sum(jnp.where(gt, ce, 0.0), axis=1, keepdims=True)
    cnt_eq = jnp.sum(eq.astype(jnp.float32), axis=1, keepdims=True)
    sum_eq = jnp.sum(jnp.where(eq, ce, 0.0), axis=1, keepdims=True)
    kf = k.astype(jnp.float32)
    neg_sum = sum_gt + (kf - cnt_gt) * sum_eq / jnp.maximum(cnt_eq, 1.0)
    pos_sum = jnp.sum(ce * posf)
    nm = jnp.sum(posf)
    loc_out[...] = (jnp.sum(lp_ref[...]) / nm).reshape(1, 1)
    conf_out[...] = ((pos_sum + jnp.sum(neg_sum)) / nm).reshape(1, 1)


@jax.jit
def kernel(loc_preds, score_preds, gt_data, priors):
    pri_t = priors.T
    locp_t = jnp.transpose(loc_preds, (0, 2, 1))

    conf, lp = pl.pallas_call(
        _match_body,
        grid=(_B,),
        in_specs=[
            pl.BlockSpec((1, _O, 5), lambda b: (b, 0, 0)),
            pl.BlockSpec((4, _P), lambda b: (0, 0)),
            pl.BlockSpec((1, 4, _P), lambda b: (b, 0, 0)),
        ],
        out_specs=[
            pl.BlockSpec((1, 1, _P), lambda b: (b, 0, 0)),
            pl.BlockSpec((1, 1, 1), lambda b: (b, 0, 0)),
        ],
        out_shape=[
            jax.ShapeDtypeStruct((_B, 1, _P), jnp.int32),
            jax.ShapeDtypeStruct((_B, 1, 1), jnp.float32),
        ],
    )(gt_data, pri_t, locp_t)
    conf = conf.reshape(_B, _P)
    lp = lp.reshape(_B, 1)

    n = _B * _P
    ce = pl.pallas_call(
        _ce_body,
        grid=(pl.cdiv(n, _CE_BLK),),
        in_specs=[
            pl.BlockSpec((_CE_BLK, _C), lambda i: (i, 0)),
            pl.BlockSpec((_CE_BLK, 1), lambda i: (i, 0)),
        ],
        out_specs=pl.BlockSpec((_CE_BLK, 1), lambda i: (i, 0)),
        out_shape=jax.ShapeDtypeStruct((n, 1), jnp.float32),
    )(score_preds.reshape(n, _C), conf.reshape(n, 1))

    loc_l, conf_l = pl.pallas_call(
        _sel_body,
        out_shape=[
            jax.ShapeDtypeStruct((1, 1), jnp.float32),
            jax.ShapeDtypeStruct((1, 1), jnp.float32),
        ],
    )(ce.reshape(_B, _P), conf, lp)
    return (loc_l[0, 0], conf_l[0, 0])


# speedup vs baseline: 12.3800x; 2.0975x over previous
import jax
import jax.numpy as jnp
from jax.experimental import pallas as pl

_B, _P, _C, _O = 32, 8732, 81, 12
_NEG_POS = 3
_OVTH = 0.5
_PB = 8736
_MR, _MC = 8, 1092
_CEW = 2048


def _match_body(gt_ref, pri_ref, locp_ref, conf_ref, lp_ref):
    px = pri_ref[0]
    py = pri_ref[1]
    pw = pri_ref[2]
    ph = pri_ref[3]
    x1 = px - pw * 0.5
    y1 = py - ph * 0.5
    x2 = px + pw * 0.5
    y2 = py + ph * 0.5
    area_p = (x2 - x1) * (y2 - y1)
    iota = (jax.lax.broadcasted_iota(jnp.int32, (_MR, _MC), 0) * _MC
            + jax.lax.broadcasted_iota(jnp.int32, (_MR, _MC), 1))

    best_ov = jnp.full((_MR, _MC), -1.0, jnp.float32)
    best_idx = jnp.zeros((_MR, _MC), jnp.int32)
    bpis = []
    for t in range(_O):
        tx1 = gt_ref[0, t, 0]
        ty1 = gt_ref[0, t, 1]
        tx2 = gt_ref[0, t, 2]
        ty2 = gt_ref[0, t, 3]
        ix = jnp.maximum(jnp.minimum(tx2, x2) - jnp.maximum(tx1, x1), 0.0)
        iy = jnp.maximum(jnp.minimum(ty2, y2) - jnp.maximum(ty1, y1), 0.0)
        inter = ix * iy
        area_t = (tx2 - tx1) * (ty2 - ty1)
        ov = inter / (area_t + area_p - inter)
        m = jnp.max(ov)
        bpis.append(jnp.min(jnp.where(ov == m, iota, _PB)))
        upd = ov > best_ov
        best_ov = jnp.where(upd, ov, best_ov)
        best_idx = jnp.where(upd, t, best_idx)
    for t in range(_O):
        hit = iota == bpis[t]
        best_ov = jnp.where(hit, 2.0, best_ov)
        best_idx = jnp.where(hit, t, best_idx)

    conf = jnp.zeros((_MR, _MC), jnp.float32)
    mx1 = jnp.zeros((_MR, _MC), jnp.float32)
    my1 = jnp.zeros((_MR, _MC), jnp.float32)
    mx2 = jnp.zeros((_MR, _MC), jnp.float32)
    my2 = jnp.zeros((_MR, _MC), jnp.float32)
    for t in range(_O):
        sel = (best_idx == t).astype(jnp.float32)
        conf = conf + sel * gt_ref[0, t, 4]
        mx1 = mx1 + sel * gt_ref[0, t, 0]
        my1 = my1 + sel * gt_ref[0, t, 1]
        mx2 = mx2 + sel * gt_ref[0, t, 2]
        my2 = my2 + sel * gt_ref[0, t, 3]
    confi = jnp.where(best_ov < _OVTH, 0, conf.astype(jnp.int32))
    conf_ref[...] = confi.reshape(1, _MR, _MC)
    posf = (confi > 0).astype(jnp.float32)

    g_cx = ((mx1 + mx2) * 0.5 - px) / (0.1 * pw)
    g_cy = ((my1 + my2) * 0.5 - py) / (0.1 * ph)
    g_w = jnp.log((mx2 - mx1) / pw) / 0.2
    g_h = jnp.log((my2 - my1) / ph) / 0.2
    s = jnp.float32(0.0)
    for d in range(4):
        df = locp_ref[0, d] - (g_cx, g_cy, g_w, g_h)[d]
        ad = jnp.abs(df)
        s = s + jnp.sum(jnp.where(ad < 1.0, 0.5 * df * df, ad - 0.5) * posf)
    lp_ref[...] = s.reshape(1, 1, 1)


def _ce_body(x_ref, tgt_ref, out_ref):
    xt = x_ref[0].T
    e = jnp.exp(xt)
    s = jnp.sum(e, axis=0, keepdims=True)
    cls = jax.lax.broadcasted_iota(jnp.int32, xt.shape, 0)
    tgt = tgt_ref[0]
    eg = jnp.sum(jnp.where(cls == tgt, e, 0.0), axis=0, keepdims=True)
    out_ref[0] = s / eg


def _sel_body(ratio_ref, conf_ref, lp_ref, loc_out, conf_out):
    r = ratio_ref[...]
    ce = jnp.log(r)
    pos = conf_ref[...] > 0
    posf = pos.astype(jnp.float32)
    nposi = jnp.sum(pos.astype(jnp.int32), axis=1, keepdims=True)
    total = jnp.sum(nposi)
    k = jnp.minimum(_NEG_POS * nposi, _P - total - 1)
    cl = jnp.where(pos, 1.0, r)
    clb = jax.lax.bitcast_convert_type(cl, jnp.int32)
    lo = jnp.zeros((_B, 1), jnp.int32)
    for bit in range(30, -1, -1):
        cand = lo + (1 << bit)
        cnt = jnp.sum((clb >= cand).astype(jnp.int32), axis=1, keepdims=True)
        lo = jnp.where(cnt >= k, cand, lo)
    gt = clb > lo
    eq = clb == lo
    cnt_gt = jnp.sum(gt.astype(jnp.float32), axis=1, keepdims=True)
    sum_gt = jnp.sum(jnp.where(gt, ce, 0.0), axis=1, keepdims=True)
    cnt_eq = jnp.sum(eq.astype(jnp.float32), axis=1, keepdims=True)
    sum_eq = jnp.sum(jnp.where(eq, ce, 0.0), axis=1, keepdims=True)
    kf = k.astype(jnp.float32)
    neg_sum = sum_gt + (kf - cnt_gt) * sum_eq / jnp.maximum(cnt_eq, 1.0)
    pos_sum = jnp.sum(ce * posf)
    nm = jnp.sum(posf)
    loc_out[...] = (jnp.sum(lp_ref[...]) / nm).reshape(1, 1)
    conf_out[...] = ((pos_sum + jnp.sum(neg_sum)) / nm).reshape(1, 1)


@jax.jit
def kernel(loc_preds, score_preds, gt_data, priors):
    pad = jnp.tile(jnp.array([[2.0, 2.0, 1.0, 1.0]], jnp.float32),
                   (_PB - _P, 1))
    pri2 = jnp.concatenate([priors, pad], axis=0).T.reshape(4, _MR, _MC)
    locp_t = jnp.transpose(loc_preds, (0, 2, 1))
    locp2 = jnp.pad(locp_t, ((0, 0), (0, 0), (0, _PB - _P))
                    ).reshape(_B, 4, _MR, _MC)

    conf8, lp = pl.pallas_call(
        _match_body,
        grid=(_B,),
        in_specs=[
            pl.BlockSpec((1, _O, 5), lambda b: (b, 0, 0)),
            pl.BlockSpec((4, _MR, _MC), lambda b: (0, 0, 0)),
            pl.BlockSpec((1, 4, _MR, _MC), lambda b: (b, 0, 0, 0)),
        ],
        out_specs=[
            pl.BlockSpec((1, _MR, _MC), lambda b: (b, 0, 0)),
            pl.BlockSpec((1, 1, 1), lambda b: (b, 0, 0)),
        ],
        out_shape=[
            jax.ShapeDtypeStruct((_B, _MR, _MC), jnp.int32),
            jax.ShapeDtypeStruct((_B, 1, 1), jnp.float32),
        ],
    )(gt_data, pri2, locp2)
    conf = conf8.reshape(_B, _PB)[:, :_P]

    nblk = pl.cdiv(_P, _CEW)
    ratio = pl.pallas_call(
        _ce_body,
        grid=(_B, nblk),
        in_specs=[
            pl.BlockSpec((1, _CEW, _C), lambda b, i: (b, i, 0)),
            pl.BlockSpec((1, 1, _CEW), lambda b, i: (b, 0, i)),
        ],
        out_specs=pl.BlockSpec((1, 1, _CEW), lambda b, i: (b, 0, i)),
        out_shape=jax.ShapeDtypeStruct((_B, 1, _P), jnp.float32),
    )(score_preds, conf.reshape(_B, 1, _P))

    loc_l, conf_l = pl.pallas_call(
        _sel_body,
        out_shape=[
            jax.ShapeDtypeStruct((1, 1), jnp.float32),
            jax.ShapeDtypeStruct((1, 1), jnp.float32),
        ],
    )(ratio.reshape(_B, _P), conf, lp.reshape(_B, 1))
    return (loc_l[0, 0], conf_l[0, 0])


# speedup vs baseline: 14.6130x; 1.1804x over previous
import jax
import jax.numpy as jnp
from jax.experimental import pallas as pl

_B, _P, _C, _O = 32, 8732, 81, 12
_NEG_POS = 3
_OVTH = 0.5
_PB = 8736
_MR, _MC = 8, 1092
_CEW = 2048


def _match_body(gt_ref, pri_ref, locp_ref, conf_ref, lp_ref):
    px = pri_ref[0:1, :]
    py = pri_ref[1:2, :]
    pw = pri_ref[2:3, :]
    ph = pri_ref[3:4, :]
    x1 = px - pw * 0.5
    y1 = py - ph * 0.5
    x2 = px + pw * 0.5
    y2 = py + ph * 0.5
    area_p = (x2 - x1) * (y2 - y1)
    g = gt_ref[0]
    tx1 = g[:, 0:1]
    ty1 = g[:, 1:2]
    tx2 = g[:, 2:3]
    ty2 = g[:, 3:4]
    lab = g[:, 4:5]
    ix = jnp.maximum(jnp.minimum(tx2, x2) - jnp.maximum(tx1, x1), 0.0)
    iy = jnp.maximum(jnp.minimum(ty2, y2) - jnp.maximum(ty1, y1), 0.0)
    inter = ix * iy
    area_t = (tx2 - tx1) * (ty2 - ty1)
    ov = inter / (area_t + area_p - inter)

    tio = jax.lax.broadcasted_iota(jnp.int32, ov.shape, 0)
    lio = jax.lax.broadcasted_iota(jnp.int32, ov.shape, 1)
    bov = jnp.max(ov, axis=0, keepdims=True)
    bidx = jnp.min(jnp.where(ov == bov, tio, _O), axis=0, keepdims=True)
    rm = jnp.max(ov, axis=1, keepdims=True)
    bpi = jnp.min(jnp.where(ov == rm, lio, _PB), axis=1, keepdims=True)
    hit = lio == bpi
    last_t = jnp.max(jnp.where(hit, tio, -1), axis=0, keepdims=True)
    anyh = last_t >= 0
    bov = jnp.where(anyh, 2.0, bov)
    bidx = jnp.where(anyh, last_t, bidx)

    sel = bidx == tio
    conf = jnp.sum(jnp.where(sel, lab, 0.0), axis=0, keepdims=True)
    mx1 = jnp.sum(jnp.where(sel, tx1, 0.0), axis=0, keepdims=True)
    my1 = jnp.sum(jnp.where(sel, ty1, 0.0), axis=0, keepdims=True)
    mx2 = jnp.sum(jnp.where(sel, tx2, 0.0), axis=0, keepdims=True)
    my2 = jnp.sum(jnp.where(sel, ty2, 0.0), axis=0, keepdims=True)
    confi = jnp.where(bov < _OVTH, 0, conf.astype(jnp.int32))
    conf_ref[...] = confi.reshape(1, 1, _PB)
    posf = (confi > 0).astype(jnp.float32)

    g_cx = ((mx1 + mx2) * 0.5 - px) / (0.1 * pw)
    g_cy = ((my1 + my2) * 0.5 - py) / (0.1 * ph)
    g_w = jnp.log((mx2 - mx1) / pw) / 0.2
    g_h = jnp.log((my2 - my1) / ph) / 0.2
    lp4 = locp_ref[0]
    s = jnp.float32(0.0)
    for d in range(4):
        df = lp4[d:d + 1, :] - (g_cx, g_cy, g_w, g_h)[d]
        ad = jnp.abs(df)
        s = s + jnp.sum(jnp.where(ad < 1.0, 0.5 * df * df, ad - 0.5) * posf)
    lp_ref[...] = s.reshape(1, 1, 1)


def _ce_body(x_ref, tgt_ref, out_ref):
    xt = x_ref[0].T
    e = jnp.exp(xt)
    s = jnp.sum(e, axis=0, keepdims=True)
    cls = jax.lax.broadcasted_iota(jnp.int32, xt.shape, 0)
    tgt = tgt_ref[0]
    eg = jnp.sum(jnp.where(cls == tgt, e, 0.0), axis=0, keepdims=True)
    out_ref[0] = s / eg


def _sel_body(ratio_ref, conf_ref, lp_ref, loc_out, conf_out):
    r = ratio_ref[...]
    ce = jnp.log(r)
    pos = conf_ref[...] > 0
    posf = pos.astype(jnp.float32)
    nposi = jnp.sum(pos.astype(jnp.int32), axis=1, keepdims=True)
    total = jnp.sum(nposi)
    k = jnp.minimum(_NEG_POS * nposi, _P - total - 1)
    cl = jnp.where(pos, 1.0, r)
    clb = jax.lax.bitcast_convert_type(cl, jnp.int32)
    lo = jnp.zeros((_B, 1), jnp.int32)
    for bit in range(30, -1, -1):
        cand = lo + (1 << bit)
        cnt = jnp.sum((clb >= cand).astype(jnp.int32), axis=1, keepdims=True)
        lo = jnp.where(cnt >= k, cand, lo)
    gt = clb > lo
    eq = clb == lo
    cnt_gt = jnp.sum(gt.astype(jnp.float32), axis=1, keepdims=True)
    sum_gt = jnp.sum(jnp.where(gt, ce, 0.0), axis=1, keepdims=True)
    cnt_eq = jnp.sum(eq.astype(jnp.float32), axis=1, keepdims=True)
    sum_eq = jnp.sum(jnp.where(eq, ce, 0.0), axis=1, keepdims=True)
    kf = k.astype(jnp.float32)
    neg_sum = sum_gt + (kf - cnt_gt) * sum_eq / jnp.maximum(cnt_eq, 1.0)
    pos_sum = jnp.sum(ce * posf)
    nm = jnp.sum(posf)
    loc_out[...] = (jnp.sum(lp_ref[...]) / nm).reshape(1, 1)
    conf_out[...] = ((pos_sum + jnp.sum(neg_sum)) / nm).reshape(1, 1)


@jax.jit
def kernel(loc_preds, score_preds, gt_data, priors):
    pad = jnp.tile(jnp.array([[2.0, 2.0, 1.0, 1.0]], jnp.float32),
                   (_PB - _P, 1))
    pri2 = jnp.concatenate([priors, pad], axis=0).T
    locp_t = jnp.transpose(loc_preds, (0, 2, 1))
    locp2 = jnp.pad(locp_t, ((0, 0), (0, 0), (0, _PB - _P)))

    conf8, lp = pl.pallas_call(
        _match_body,
        grid=(_B,),
        in_specs=[
            pl.BlockSpec((1, _O, 5), lambda b: (b, 0, 0)),
            pl.BlockSpec((4, _PB), lambda b: (0, 0)),
            pl.BlockSpec((1, 4, _PB), lambda b: (b, 0, 0)),
        ],
        out_specs=[
            pl.BlockSpec((1, 1, _PB), lambda b: (b, 0, 0)),
            pl.BlockSpec((1, 1, 1), lambda b: (b, 0, 0)),
        ],
        out_shape=[
            jax.ShapeDtypeStruct((_B, 1, _PB), jnp.int32),
            jax.ShapeDtypeStruct((_B, 1, 1), jnp.float32),
        ],
    )(gt_data, pri2, locp2)
    conf = conf8.reshape(_B, _PB)[:, :_P]

    nblk = pl.cdiv(_P, _CEW)
    ratio = pl.pallas_call(
        _ce_body,
        grid=(_B, nblk),
        in_specs=[
            pl.BlockSpec((1, _CEW, _C), lambda b, i: (b, i, 0)),
            pl.BlockSpec((1, 1, _CEW), lambda b, i: (b, 0, i)),
        ],
        out_specs=pl.BlockSpec((1, 1, _CEW), lambda b, i: (b, 0, i)),
        out_shape=jax.ShapeDtypeStruct((_B, 1, _P), jnp.float32),
    )(score_preds, conf.reshape(_B, 1, _P))

    loc_l, conf_l = pl.pallas_call(
        _sel_body,
        out_shape=[
            jax.ShapeDtypeStruct((1, 1), jnp.float32),
            jax.ShapeDtypeStruct((1, 1), jnp.float32),
        ],
    )(ratio.reshape(_B, _P), conf, lp.reshape(_B, 1))
    return (loc_l[0, 0], conf_l[0, 0])
